# Initial kernel scaffold; baseline (speedup 1.0000x reference)
import jax
import jax.numpy as jnp
from jax.experimental import pallas as pl


def kernel(input, weight, weight_scale):
    raise NotImplementedError("write your pallas kernel here")


# speedup vs baseline: 5.4352x; 5.4352x over previous
import functools

import jax
import jax.numpy as jnp
from jax import lax
from jax.experimental import pallas as pl
from jax.experimental.pallas import tpu as pltpu
from jax.experimental.pallas import tpu_sc as plsc

NC = 2
NS = 16
CHUNK = 128


def _build(n_rows, emb_dim, n_chunks_per_tile):
    nw = NC * NS
    mesh = plsc.VectorSubcoreMesh(core_axis_name="c", subcore_axis_name="s")
    words_per_row = emb_dim // 4

    @functools.partial(
        pl.kernel,
        mesh=mesh,
        out_type=jax.ShapeDtypeStruct((n_rows, emb_dim), jnp.float32),
        scratch_types=[
            pltpu.VMEM((n_chunks_per_tile, CHUNK), jnp.int32),
            pltpu.VMEM((CHUNK, words_per_row), jnp.int32),
            pltpu.VMEM((CHUNK, emb_dim), jnp.float32),
            pltpu.VMEM((4, 16), jnp.float32),
            pltpu.SemaphoreType.DMA,
        ],
        compiler_params=pltpu.CompilerParams(needs_layout_passes=False,
                                             use_tc_tiling_on_sc=False),
    )
    def body(idx_hbm, table_hbm, scale_hbm, out_hbm, idx_v, rows_v, out_v,
             scale_v, sem):
        wid = lax.axis_index("s") * NC + lax.axis_index("c")
        pltpu.sync_copy(scale_hbm, scale_v)
        pltpu.sync_copy(
            idx_hbm.at[pl.ds(wid * n_chunks_per_tile, n_chunks_per_tile)],
            idx_v)

        lane = lax.iota(jnp.int32, 16)
        row_pat = lane // 8
        word_idx = lane % 8
        ch_base = 4 * (lane % 8)
        scales = [scale_v[k, :] for k in range(4)]
        tile_base = wid * (n_chunks_per_tile * CHUNK)

        def do_chunk(j, _):
            pltpu.async_copy(table_hbm.at[idx_v.at[j]], rows_v, sem).wait()

            def dequant2(i, _):
                row_idx = row_pat + 2 * i
                x = plsc.load_gather(rows_v, [row_idx, word_idx])
                for k in range(4):
                    v = (x << (24 - 8 * k)) >> 24
                    y = v.astype(jnp.float32) * scales[k]
                    plsc.store_scatter(out_v, [row_idx, ch_base + k], y)
                return 0

            lax.fori_loop(0, CHUNK // 2, dequant2, 0)
            pltpu.sync_copy(out_v,
                            out_hbm.at[pl.ds(tile_base + j * CHUNK, CHUNK)])
            return 0

        lax.fori_loop(0, n_chunks_per_tile, do_chunk, 0)

    return body


def kernel(input, weight, weight_scale):
    batch, n_fields = input.shape
    n_rows = batch * n_fields
    emb_dim = weight.shape[1]
    nw = NC * NS
    n_chunks_per_tile = n_rows // (nw * CHUNK)

    idx = input.reshape(nw * n_chunks_per_tile, CHUNK)
    table_i32 = lax.bitcast_convert_type(
        weight.reshape(weight.shape[0], emb_dim // 4, 4), jnp.int32)
    l = jnp.arange(16)
    k = jnp.arange(4)
    scale_perm = weight_scale[4 * (l[None, :] % 8) + k[:, None]]

    out = _build(n_rows, emb_dim, n_chunks_per_tile)(idx, table_i32,
                                                     scale_perm)
    return out.reshape(batch, n_fields, emb_dim)


# speedup vs baseline: 8.1399x; 1.4976x over previous
import functools

import jax
import jax.numpy as jnp
from jax import lax
from jax.experimental import pallas as pl
from jax.experimental.pallas import tpu as pltpu
from jax.experimental.pallas import tpu_sc as plsc

NC = 2
NS = 16
CHUNK = 128


@functools.cache
def _build(n_rows, emb_dim, n_chunks_per_tile):
    nw = NC * NS
    mesh = plsc.VectorSubcoreMesh(core_axis_name="c", subcore_axis_name="s")

    @functools.partial(
        pl.kernel,
        mesh=mesh,
        out_type=jax.ShapeDtypeStruct((n_rows, emb_dim), jnp.float32),
        scratch_types=[
            pltpu.VMEM((n_chunks_per_tile, CHUNK), jnp.int32),
            pltpu.VMEM((CHUNK, emb_dim), jnp.int8),
            pltpu.VMEM((CHUNK, emb_dim), jnp.float32),
            pltpu.VMEM((4, 16), jnp.float32),
            pltpu.SemaphoreType.DMA,
        ],
        compiler_params=pltpu.CompilerParams(needs_layout_passes=False,
                                             use_tc_tiling_on_sc=False),
    )
    def body(idx_hbm, table_hbm, scale_hbm, out_hbm, idx_v, rows_v, out_v,
             scale_v, sem):
        wid = lax.axis_index("s") * NC + lax.axis_index("c")
        pltpu.sync_copy(scale_hbm, scale_v)
        pltpu.sync_copy(
            idx_hbm.at[pl.ds(wid * n_chunks_per_tile, n_chunks_per_tile)],
            idx_v)

        lane = lax.iota(jnp.int32, 16)
        row_pat = lane // 8
        ch_base = 4 * (lane % 8)
        scales = [scale_v[k, :] for k in range(4)]
        rows32 = rows_v.bitcast(jnp.int32)
        tile_base = wid * (n_chunks_per_tile * CHUNK)

        def do_chunk(j, _):
            pltpu.async_copy(table_hbm.at[idx_v.at[j]], rows_v, sem).wait()

            def dequant2(i, _):
                row_idx = row_pat + 2 * i
                x = rows32[i // 2, pl.ds((i % 2) * 16, 16)]
                for k in range(4):
                    v = (x << (24 - 8 * k)) >> 24
                    y = v.astype(jnp.float32) * scales[k]
                    plsc.store_scatter(out_v, [row_idx, ch_base + k], y)
                return 0

            lax.fori_loop(0, CHUNK // 2, dequant2, 0)
            pltpu.sync_copy(out_v,
                            out_hbm.at[pl.ds(tile_base + j * CHUNK, CHUNK)])
            return 0

        lax.fori_loop(0, n_chunks_per_tile, do_chunk, 0)

    return body


def kernel(input, weight, weight_scale):
    batch, n_fields = input.shape
    n_rows = batch * n_fields
    emb_dim = weight.shape[1]
    nw = NC * NS
    n_chunks_per_tile = n_rows // (nw * CHUNK)

    idx = input.reshape(nw * n_chunks_per_tile, CHUNK)
    l = jnp.arange(16)
    k = jnp.arange(4)
    scale_perm = weight_scale[4 * (l[None, :] % 8) + k[:, None]]

    out = _build(n_rows, emb_dim, n_chunks_per_tile)(idx, weight, scale_perm)
    return out.reshape(batch, n_fields, emb_dim)
